# baseline (speedup 1.0000x reference)
import functools

import jax
import jax.numpy as jnp
from jax import lax
from jax.experimental import pallas as pl
from jax.experimental.pallas import tpu as pltpu
from jax.experimental.pallas import tpu_sc as plsc

VOCAB = 1000000
D = 64
DP = 128
BATCH = 4096
SEQ = 200
B = BATCH * SEQ
NC = 2
NS = 16
NW = NC * NS
B_PER_W = B // NW
C = 40
NCH = B_PER_W // C
NSLOT = 8
LOOK = 4


def _make_gather():
    mesh = plsc.VectorSubcoreMesh(core_axis_name="c", subcore_axis_name="s")

    @functools.partial(
        pl.kernel,
        mesh=mesh,
        out_type=jax.ShapeDtypeStruct((BATCH, SEQ, D), jnp.float32),
        scratch_types=[
            pltpu.VMEM((NCH, C), jnp.int32),
            pltpu.VMEM((NSLOT, C, D), jnp.float32),
            [pltpu.SemaphoreType.DMA] * NSLOT,
            [pltpu.SemaphoreType.DMA] * NSLOT,
        ],
        compiler_params=pltpu.CompilerParams(use_tc_tiling_on_sc=False),
    )
    def gather_kernel(x_hbm, table_hbm, out_hbm, idx_v, rows_v, gsems, ssems):
        wid = lax.axis_index("s") * NC + lax.axis_index("c")
        bb0 = wid * (BATCH // NW)
        pltpu.sync_copy(x_hbm.at[wid], idx_v)

        def gather_chunk(j, slot):
            pltpu.async_copy(
                table_hbm.at[idx_v.at[j]], rows_v.at[slot], gsems[slot]
            )

        def store_copy(j, slot, sem):
            return pltpu.make_async_copy(
                rows_v.at[slot],
                out_hbm.at[bb0 + j // 5, pl.ds((j % 5) * C, C)],
                sem,
            )

        for j in range(LOOK):
            gather_chunk(j, j % NSLOT)

        def group_body(g, carry):
            for bi in range(NSLOT):
                j = g * NSLOT + bi
                pltpu.make_async_copy(
                    table_hbm.at[idx_v.at[j]], rows_v.at[bi], gsems[bi]
                ).wait()
                store_copy(j, bi, ssems[bi]).start()
                jf = j + LOOK
                sf = (bi + LOOK) % NSLOT

                @pl.when(jf < NCH)
                def _():
                    @pl.when(jf >= NSLOT)
                    def _():
                        store_copy(jf - NSLOT, sf, ssems[sf]).wait()

                    gather_chunk(jf, sf)

            return carry

        lax.fori_loop(0, NCH // NSLOT, group_body, 0)

        for bi in range(NSLOT):
            j = NCH - NSLOT + bi
            store_copy(j, bi, ssems[bi]).wait()

    return gather_kernel


_gather = _make_gather()


def kernel(x, embedding_table):
    idx = x.reshape(NW, NCH, C)
    emb = _gather(idx, embedding_table)
    return (emb, emb)


# speedup vs baseline: 1.3104x; 1.3104x over previous
import functools

import jax
import jax.numpy as jnp
from jax import lax
from jax.experimental import pallas as pl
from jax.experimental.pallas import tpu as pltpu
from jax.experimental.pallas import tpu_sc as plsc

VOCAB = 1000000
D = 64
DP = 128
BATCH = 4096
SEQ = 200
B = BATCH * SEQ
NC = 2
NS = 16
NW = NC * NS
B_PER_W = B // NW
C = 128
NCH = B_PER_W // C
NSLOT = 8
LOOK = 4


def _make_gather():
    mesh = plsc.VectorSubcoreMesh(core_axis_name="c", subcore_axis_name="s")

    @functools.partial(
        pl.kernel,
        mesh=mesh,
        out_type=jax.ShapeDtypeStruct((B, DP), jnp.float32),
        scratch_types=[
            pltpu.VMEM((NCH, C), jnp.int32),
            pltpu.VMEM((NSLOT, C, D), jnp.float32),
            [pltpu.SemaphoreType.DMA] * NSLOT,
            [pltpu.SemaphoreType.DMA] * NSLOT,
        ],
        compiler_params=pltpu.CompilerParams(use_tc_tiling_on_sc=False),
    )
    def gather_kernel(x_hbm, table_hbm, out_hbm, idx_v, rows_v, gsems, ssems):
        wid = lax.axis_index("s") * NC + lax.axis_index("c")
        base0 = wid * B_PER_W
        pltpu.sync_copy(x_hbm.at[wid], idx_v)

        def gather_chunk(j, slot):
            pltpu.async_copy(
                table_hbm.at[idx_v.at[j]], rows_v.at[slot], gsems[slot]
            )

        def store_copy(j, slot, sem):
            return pltpu.make_async_copy(
                rows_v.at[slot],
                out_hbm.at[pl.ds((base0 + j * C) , C), pl.ds(0, D)],
                sem,
            )

        for j in range(LOOK):
            gather_chunk(j, j % NSLOT)

        def group_body(g, carry):
            for bi in range(NSLOT):
                j = g * NSLOT + bi
                pltpu.make_async_copy(
                    table_hbm.at[idx_v.at[j]], rows_v.at[bi], gsems[bi]
                ).wait()
                store_copy(j, bi, ssems[bi]).start()
                jf = j + LOOK
                sf = (bi + LOOK) % NSLOT

                @pl.when(jf < NCH)
                def _():
                    @pl.when(jf >= NSLOT)
                    def _():
                        store_copy(jf - NSLOT, sf, ssems[sf]).wait()

                    gather_chunk(jf, sf)

            return carry

        lax.fori_loop(0, NCH // NSLOT, group_body, 0)

        for bi in range(NSLOT):
            j = NCH - NSLOT + bi
            store_copy(j, bi, ssems[bi]).wait()

    return gather_kernel


_gather = _make_gather()


def kernel(x, embedding_table):
    idx = x.reshape(NW, NCH, C)
    emb = _gather(idx, embedding_table)[:, :D].reshape(BATCH, SEQ, D)
    return (emb, emb)


# speedup vs baseline: 1.3122x; 1.0014x over previous
import functools

import jax
import jax.numpy as jnp
from jax import lax
from jax.experimental import pallas as pl
from jax.experimental.pallas import tpu as pltpu
from jax.experimental.pallas import tpu_sc as plsc

VOCAB = 1000000
D = 64
DP = 128
BATCH = 4096
SEQ = 200
B = BATCH * SEQ
NC = 2
NS = 16
NW = NC * NS
B_PER_W = B // NW
C = 128
NCH = B_PER_W // C
NSLOT = 8
LOOK = 6


def _make_gather():
    mesh = plsc.VectorSubcoreMesh(core_axis_name="c", subcore_axis_name="s")

    @functools.partial(
        pl.kernel,
        mesh=mesh,
        out_type=jax.ShapeDtypeStruct((B, DP), jnp.float32),
        scratch_types=[
            pltpu.VMEM((NCH, C), jnp.int32),
            pltpu.VMEM((NSLOT, C, D), jnp.float32),
            [pltpu.SemaphoreType.DMA] * NSLOT,
            [pltpu.SemaphoreType.DMA] * NSLOT,
        ],
        compiler_params=pltpu.CompilerParams(use_tc_tiling_on_sc=False),
    )
    def gather_kernel(x_hbm, table_hbm, out_hbm, idx_v, rows_v, gsems, ssems):
        wid = lax.axis_index("s") * NC + lax.axis_index("c")
        base0 = wid * B_PER_W
        pltpu.sync_copy(x_hbm.at[wid], idx_v)

        def gather_chunk(j, slot):
            pltpu.async_copy(
                table_hbm.at[idx_v.at[j]], rows_v.at[slot], gsems[slot]
            )

        def store_copy(j, slot, sem):
            return pltpu.make_async_copy(
                rows_v.at[slot],
                out_hbm.at[pl.ds((base0 + j * C) , C), pl.ds(0, D)],
                sem,
            )

        for j in range(LOOK):
            gather_chunk(j, j % NSLOT)

        def group_body(g, carry):
            for bi in range(NSLOT):
                j = g * NSLOT + bi
                pltpu.make_async_copy(
                    table_hbm.at[idx_v.at[j]], rows_v.at[bi], gsems[bi]
                ).wait()
                store_copy(j, bi, ssems[bi]).start()
                jf = j + LOOK
                sf = (bi + LOOK) % NSLOT

                @pl.when(jf < NCH)
                def _():
                    @pl.when(jf >= NSLOT)
                    def _():
                        store_copy(jf - NSLOT, sf, ssems[sf]).wait()

                    gather_chunk(jf, sf)

            return carry

        lax.fori_loop(0, NCH // NSLOT, group_body, 0)

        for bi in range(NSLOT):
            j = NCH - NSLOT + bi
            store_copy(j, bi, ssems[bi]).wait()

    return gather_kernel


_gather = _make_gather()


def kernel(x, embedding_table):
    idx = x.reshape(NW, NCH, C)
    emb = _gather(idx, embedding_table)[:, :D].reshape(BATCH, SEQ, D)
    return (emb, emb)
